# baseline (speedup 1.0000x reference)
import functools

import jax
import jax.numpy as jnp
from jax import lax
from jax.experimental import pallas as pl
from jax.experimental.pallas import tpu as pltpu
from jax.experimental.pallas import tpu_sc as plsc

N = 10000
E = 320000
D = 128
BN_EPS = 1e-5

NC = 2
NS = 16
NW = NC * NS
CHUNK = 100
N_CHUNKS = E // CHUNK
NCH = N_CHUNKS // NW
ACH = 128
ANCH = 79
A_CHUNKS = NW * ANCH
E_PAD = A_CHUNKS * ACH
NBUF = 4
ROWS_PER_TILE = N // NS

_mesh = plsc.VectorSubcoreMesh(core_axis_name="c", subcore_axis_name="s")
_sc_params = pltpu.CompilerParams(use_tc_tiling_on_sc=False)


def _worker_id():
    cid = lax.axis_index("c")
    sid = lax.axis_index("s")
    return sid * NC + cid, cid, sid


@functools.partial(
    pl.kernel,
    out_type=jax.ShapeDtypeStruct((NC, N, 16), jnp.float32),
    mesh=_mesh,
    compiler_params=_sc_params,
    scratch_types=[
        pltpu.VMEM((NCH, CHUNK), jnp.int32),
        pltpu.VMEM((CHUNK, 16), jnp.float32),
        pltpu.VMEM((ROWS_PER_TILE, 16), jnp.float32),
        pltpu.VMEM_SHARED((N, 16), jnp.float32),
        [pltpu.SemaphoreType.DMA for _ in range(4)],
    ],
)
def _sc_degree(col2d_hbm, deg_hbm, cbuf, ones_v, zeros_v, acc_sh, ds_sems):
    wid, cid, sid = _worker_id()
    pltpu.sync_copy(col2d_hbm.at[pl.ds(wid * NCH, NCH)], cbuf)

    def fill(i, _):
        ones_v[i, :] = jnp.ones((16,), jnp.float32)
        return 0

    lax.fori_loop(0, CHUNK, fill, 0)

    def zfill(i, _):
        zeros_v[i, :] = jnp.zeros((16,), jnp.float32)
        return 0

    lax.fori_loop(0, ROWS_PER_TILE, zfill, 0)
    pltpu.sync_copy(zeros_v, acc_sh.at[pl.ds(sid * ROWS_PER_TILE, ROWS_PER_TILE)])
    plsc.subcore_barrier()

    def dscat(k, b):
        return pltpu.async_copy(ones_v, acc_sh.at[cbuf.at[k]], ds_sems[b],
                                add=True)

    def dscat_wait(k, b):
        pltpu.make_async_copy(ones_v, acc_sh.at[cbuf.at[k]], ds_sems[b]).wait()

    for j in range(4):
        dscat(j, j)

    def body(i, _):
        for off in range(4):
            j = 4 * i + off
            dscat_wait(j - 4, off)
            dscat(j, off)
        return 0

    lax.fori_loop(1, NCH // 4, body, 0)
    for off in range(4):
        dscat_wait(NCH - 4 + off, off)
    plsc.subcore_barrier()
    pltpu.sync_copy(
        acc_sh.at[pl.ds(sid * ROWS_PER_TILE, ROWS_PER_TILE)],
        deg_hbm.at[cid, pl.ds(sid * ROWS_PER_TILE, ROWS_PER_TILE)],
    )


@functools.partial(
    pl.kernel,
    out_type=jax.ShapeDtypeStruct((NC, N, D), jnp.bfloat16),
    mesh=_mesh,
    compiler_params=_sc_params,
    scratch_types=[
        pltpu.VMEM((ANCH, ACH), jnp.int32),
        pltpu.VMEM((ANCH, ACH), jnp.int32),
        [pltpu.VMEM((ACH, D), jnp.bfloat16) for _ in range(NBUF)],
        pltpu.VMEM_SHARED((N + ACH, D), jnp.bfloat16),
        [pltpu.SemaphoreType.DMA for _ in range(NBUF)],
        [pltpu.SemaphoreType.DMA for _ in range(NBUF)],
    ],
)
def _sc_aggregate(g_hbm, row2d_hbm, col2d_hbm, out_hbm,
                  rbuf, cbuf, rows, acc_sh, sg, ss):
    wid, cid, sid = _worker_id()
    pltpu.sync_copy(row2d_hbm.at[pl.ds(wid * ANCH, ANCH)], rbuf)
    pltpu.sync_copy(col2d_hbm.at[pl.ds(wid * ANCH, ANCH)], cbuf)

    def zfill(i, _):
        for j in range(D // 32):
            rows[0][i, pl.ds(j * 32, 32)] = jnp.zeros((32,), jnp.bfloat16)
        return 0

    lax.fori_loop(0, ACH, zfill, 0)

    def zcopy(k, _):
        pltpu.sync_copy(
            rows[0], acc_sh.at[pl.ds(sid * ROWS_PER_TILE + k * ACH, ACH)]
        )
        return 0

    lax.fori_loop(0, ROWS_PER_TILE // ACH, zcopy, 0)
    pltpu.sync_copy(
        rows[0].at[pl.ds(0, ROWS_PER_TILE % ACH)],
        acc_sh.at[pl.ds(sid * ROWS_PER_TILE + (ROWS_PER_TILE // ACH) * ACH,
                        ROWS_PER_TILE % ACH)],
    )
    plsc.subcore_barrier()

    def gather(k, b):
        pltpu.async_copy(g_hbm.at[rbuf.at[k]], rows[b], sg[b])

    def gather_wait(k, b):
        pltpu.make_async_copy(g_hbm.at[rbuf.at[k]], rows[b], sg[b]).wait()

    def scatter(k, b):
        pltpu.async_copy(rows[b], acc_sh.at[cbuf.at[k]], ss[b], add=True)

    def scatter_wait(k, b):
        pltpu.make_async_copy(rows[b], acc_sh.at[cbuf.at[k]], ss[b]).wait()

    def slot(j, b):
        gather_wait(j, b)
        scatter(j, b)
        if isinstance(j, int):
            if j >= 2:
                scatter_wait(j - 2, (b - 2) % NBUF)
            if j + 2 < ANCH:
                gather(j + 2, (b + 2) % NBUF)
        else:
            scatter_wait(j - 2, (b - 2) % NBUF)
            gather(j + 2, (b + 2) % NBUF)

    gather(0, 0)
    gather(1, 1)
    slot(0, 0)
    slot(1, 1)

    def quad(i, _):
        for off in range(NBUF):
            j = 2 + NBUF * i + off
            slot(j, (2 + off) % NBUF)
        return 0

    lax.fori_loop(0, 18, quad, 0)
    for j in range(74, ANCH):
        slot(j, j % NBUF)
    scatter_wait(ANCH - 2, (ANCH - 2) % NBUF)
    scatter_wait(ANCH - 1, (ANCH - 1) % NBUF)
    plsc.subcore_barrier()
    pltpu.sync_copy(
        acc_sh.at[pl.ds(sid * ROWS_PER_TILE, ROWS_PER_TILE)],
        out_hbm.at[cid, pl.ds(sid * ROWS_PER_TILE, ROWS_PER_TILE)],
    )


ROW_BLOCK = 1000


def _tc_linear_body(deg_ref, x_ref, w_ref, g_ref):
    deg = deg_ref[0, :, 0] + deg_ref[1, :, 0] + 1.0
    dinv = lax.rsqrt(deg)
    h = jnp.dot(x_ref[...], w_ref[...], preferred_element_type=jnp.float32)
    g_ref[...] = (h * dinv[:, None]).astype(jnp.bfloat16)


def _tc_linear(deg, x, W):
    grid = N // ROW_BLOCK
    return pl.pallas_call(
        _tc_linear_body,
        grid=(grid,),
        in_specs=[
            pl.BlockSpec((NC, ROW_BLOCK, 16), lambda i: (0, i, 0)),
            pl.BlockSpec((ROW_BLOCK, D), lambda i: (i, 0)),
            pl.BlockSpec((D, D), lambda i: (0, 0)),
        ],
        out_specs=pl.BlockSpec((ROW_BLOCK, D), lambda i: (i, 0)),
        out_shape=jax.ShapeDtypeStruct((N, D), jnp.bfloat16),
    )(deg, x, W)


def _tc_final_body(part_ref, g_ref, deg_ref, b_ref, gam_ref, bet_ref, o_ref):
    deg = deg_ref[0, :, 0] + deg_ref[1, :, 0] + 1.0
    dinv = lax.rsqrt(deg)
    s = (part_ref[0].astype(jnp.float32) + part_ref[1].astype(jnp.float32)
         + g_ref[...].astype(jnp.float32))
    scale = (1.0 / jnp.sqrt(1.0 + BN_EPS))
    o = (s * dinv[:, None] + b_ref[0]) * (gam_ref[0] * scale) + bet_ref[0]
    o_ref[...] = jnp.maximum(o, 0.0)


def _tc_final(part, g, deg, b, gamma, beta):
    grid = N // ROW_BLOCK
    return pl.pallas_call(
        _tc_final_body,
        grid=(grid,),
        in_specs=[
            pl.BlockSpec((NC, ROW_BLOCK, D), lambda i: (0, i, 0)),
            pl.BlockSpec((ROW_BLOCK, D), lambda i: (i, 0)),
            pl.BlockSpec((NC, ROW_BLOCK, 16), lambda i: (0, i, 0)),
            pl.BlockSpec((1, D), lambda i: (0, 0)),
            pl.BlockSpec((1, D), lambda i: (0, 0)),
            pl.BlockSpec((1, D), lambda i: (0, 0)),
        ],
        out_specs=pl.BlockSpec((ROW_BLOCK, D), lambda i: (i, 0)),
        out_shape=jax.ShapeDtypeStruct((N, D), jnp.float32),
    )(part, g, deg, b, gamma, beta)


def kernel(x, edge_index, W, b, gamma, beta):
    row32 = edge_index[0].astype(jnp.int32)
    col32 = edge_index[1].astype(jnp.int32)
    assert N_CHUNKS == NW * NCH and E_PAD == A_CHUNKS * ACH
    npad = E_PAD - E
    row_pad = jnp.concatenate([row32, jnp.zeros((npad,), jnp.int32)])
    col_pad = jnp.concatenate(
        [col32, N + (jnp.arange(npad, dtype=jnp.int32) % ACH)])
    deg = _sc_degree(col32.reshape(N_CHUNKS, CHUNK))
    g = _tc_linear(deg, x, W)
    part = _sc_aggregate(g, row_pad.reshape(A_CHUNKS, ACH),
                         col_pad.reshape(A_CHUNKS, ACH))
    return _tc_final(part, g, deg, b.reshape(1, D), gamma.reshape(1, D),
                     beta.reshape(1, D))


# speedup vs baseline: 1.4261x; 1.4261x over previous
import functools

import jax
import jax.numpy as jnp
from jax import lax
from jax.experimental import pallas as pl
from jax.experimental.pallas import tpu as pltpu
from jax.experimental.pallas import tpu_sc as plsc

N = 10000
E = 320000
D = 128
BN_EPS = 1e-5

NC = 2
NS = 16
NW = NC * NS
CHUNK = 100
N_CHUNKS = E // CHUNK
NCH = N_CHUNKS // NW
ACH = 128
ANCH = 79
A_CHUNKS = NW * ANCH
E_PAD = A_CHUNKS * ACH
NBUF = 4
ROWS_PER_TILE = N // NS

_mesh = plsc.VectorSubcoreMesh(core_axis_name="c", subcore_axis_name="s")
_sc_params = pltpu.CompilerParams(use_tc_tiling_on_sc=False)


def _worker_id():
    cid = lax.axis_index("c")
    sid = lax.axis_index("s")
    return sid * NC + cid, cid, sid


@functools.partial(
    pl.kernel,
    out_type=jax.ShapeDtypeStruct((NC, N, 16), jnp.float32),
    mesh=_mesh,
    compiler_params=_sc_params,
    scratch_types=[
        pltpu.VMEM((NCH, CHUNK), jnp.int32),
        pltpu.VMEM((CHUNK, 16), jnp.float32),
        pltpu.VMEM((ROWS_PER_TILE, 16), jnp.float32),
        pltpu.VMEM_SHARED((N, 16), jnp.float32),
        [pltpu.SemaphoreType.DMA for _ in range(4)],
    ],
)
def _sc_degree(col2d_hbm, deg_hbm, cbuf, ones_v, zeros_v, acc_sh, ds_sems):
    wid, cid, sid = _worker_id()
    pltpu.sync_copy(col2d_hbm.at[pl.ds(wid * NCH, NCH)], cbuf)

    def fill(i, _):
        ones_v[i, :] = jnp.ones((16,), jnp.float32)
        return 0

    lax.fori_loop(0, CHUNK, fill, 0)

    def zfill(i, _):
        zeros_v[i, :] = jnp.zeros((16,), jnp.float32)
        return 0

    lax.fori_loop(0, ROWS_PER_TILE, zfill, 0)
    pltpu.sync_copy(zeros_v, acc_sh.at[pl.ds(sid * ROWS_PER_TILE, ROWS_PER_TILE)])
    plsc.subcore_barrier()

    def dscat(k, b):
        return pltpu.async_copy(ones_v, acc_sh.at[cbuf.at[k]], ds_sems[b],
                                add=True)

    def dscat_wait(k, b):
        pltpu.make_async_copy(ones_v, acc_sh.at[cbuf.at[k]], ds_sems[b]).wait()

    for j in range(4):
        dscat(j, j)

    def body(i, _):
        for off in range(4):
            j = 4 * i + off
            dscat_wait(j - 4, off)
            dscat(j, off)
        return 0

    lax.fori_loop(1, NCH // 4, body, 0)
    for off in range(4):
        dscat_wait(NCH - 4 + off, off)
    plsc.subcore_barrier()
    pltpu.sync_copy(
        acc_sh.at[pl.ds(sid * ROWS_PER_TILE, ROWS_PER_TILE)],
        deg_hbm.at[cid, pl.ds(sid * ROWS_PER_TILE, ROWS_PER_TILE)],
    )


@functools.partial(
    pl.kernel,
    out_type=jax.ShapeDtypeStruct((NC, N, D), jnp.bfloat16),
    mesh=_mesh,
    compiler_params=_sc_params,
    scratch_types=[
        pltpu.VMEM((ANCH, ACH), jnp.int32),
        pltpu.VMEM((ANCH, ACH), jnp.int32),
        [pltpu.VMEM((ACH, D), jnp.bfloat16) for _ in range(NBUF)],
        pltpu.VMEM_SHARED((N + ACH, D), jnp.bfloat16),
        [pltpu.SemaphoreType.DMA for _ in range(NBUF)],
        [pltpu.SemaphoreType.DMA for _ in range(NBUF)],
    ],
)
def _sc_aggregate(g_hbm, row2d_hbm, col2d_hbm, out_hbm,
                  rbuf, cbuf, rows, acc_sh, sg, ss):
    wid, cid, sid = _worker_id()
    pltpu.sync_copy(row2d_hbm.at[pl.ds(wid * ANCH, ANCH)], rbuf)
    pltpu.sync_copy(col2d_hbm.at[pl.ds(wid * ANCH, ANCH)], cbuf)

    def zfill(i, _):
        for j in range(D // 32):
            rows[0][i, pl.ds(j * 32, 32)] = jnp.zeros((32,), jnp.bfloat16)
        return 0

    lax.fori_loop(0, ACH, zfill, 0)

    def zcopy(k, _):
        pltpu.sync_copy(
            rows[0], acc_sh.at[pl.ds(sid * ROWS_PER_TILE + k * ACH, ACH)]
        )
        return 0

    lax.fori_loop(0, ROWS_PER_TILE // ACH, zcopy, 0)
    pltpu.sync_copy(
        rows[0].at[pl.ds(0, ROWS_PER_TILE % ACH)],
        acc_sh.at[pl.ds(sid * ROWS_PER_TILE + (ROWS_PER_TILE // ACH) * ACH,
                        ROWS_PER_TILE % ACH)],
    )
    plsc.subcore_barrier()

    def gather(k, b):
        pltpu.async_copy(g_hbm.at[rbuf.at[k]], rows[b], sg[b])

    def gather_wait(k, b):
        pltpu.make_async_copy(g_hbm.at[rbuf.at[k]], rows[b], sg[b]).wait()

    def scatter(k, b):
        pltpu.async_copy(rows[b], acc_sh.at[cbuf.at[k]], ss[b], add=True)

    def scatter_wait(k, b):
        pltpu.make_async_copy(rows[b], acc_sh.at[cbuf.at[k]], ss[b]).wait()

    def slot(j, b):
        gather_wait(j, b)
        scatter(j, b)
        if isinstance(j, int):
            if j >= 2:
                scatter_wait(j - 2, (b - 2) % NBUF)
            if j + 2 < ANCH:
                gather(j + 2, (b + 2) % NBUF)
        else:
            scatter_wait(j - 2, (b - 2) % NBUF)
            gather(j + 2, (b + 2) % NBUF)

    gather(0, 0)
    gather(1, 1)
    slot(0, 0)
    slot(1, 1)

    def quad(i, _):
        for off in range(NBUF):
            j = 2 + NBUF * i + off
            slot(j, (2 + off) % NBUF)
        return 0

    lax.fori_loop(0, 18, quad, 0)
    for j in range(74, ANCH):
        slot(j, j % NBUF)
    scatter_wait(ANCH - 2, (ANCH - 2) % NBUF)
    scatter_wait(ANCH - 1, (ANCH - 1) % NBUF)
    plsc.subcore_barrier()
    pltpu.sync_copy(
        acc_sh.at[pl.ds(sid * ROWS_PER_TILE, ROWS_PER_TILE)],
        out_hbm.at[cid, pl.ds(sid * ROWS_PER_TILE, ROWS_PER_TILE)],
    )


ROW_BLOCK = 1000


def _tc_linear_body(deg_ref, x_ref, w_ref, g_ref):
    deg = deg_ref[0, :, 0] + deg_ref[1, :, 0] + 1.0
    dinv = lax.rsqrt(deg)
    h = jnp.dot(x_ref[...], w_ref[...], preferred_element_type=jnp.float32)
    g_ref[...] = (h * dinv[:, None]).astype(jnp.bfloat16)


def _tc_linear(deg, x, W):
    grid = N // ROW_BLOCK
    return pl.pallas_call(
        _tc_linear_body,
        grid=(grid,),
        in_specs=[
            pl.BlockSpec((NC, ROW_BLOCK, 16), lambda i: (0, i, 0)),
            pl.BlockSpec((ROW_BLOCK, D), lambda i: (i, 0)),
            pl.BlockSpec((D, D), lambda i: (0, 0)),
        ],
        out_specs=pl.BlockSpec((ROW_BLOCK, D), lambda i: (i, 0)),
        out_shape=jax.ShapeDtypeStruct((N, D), jnp.bfloat16),
    )(deg, x, W)


def _tc_final_body(part_ref, g_ref, deg_ref, b_ref, gam_ref, bet_ref, o_ref):
    deg = deg_ref[0, :, 0] + deg_ref[1, :, 0] + 1.0
    dinv = lax.rsqrt(deg)
    s = (part_ref[0].astype(jnp.float32) + part_ref[1].astype(jnp.float32)
         + g_ref[...].astype(jnp.float32))
    scale = (1.0 / jnp.sqrt(1.0 + BN_EPS))
    o = (s * dinv[:, None] + b_ref[0]) * (gam_ref[0] * scale) + bet_ref[0]
    o_ref[...] = jnp.maximum(o, 0.0)


def _tc_final(part, g, deg, b, gamma, beta):
    grid = N // ROW_BLOCK
    return pl.pallas_call(
        _tc_final_body,
        grid=(grid,),
        in_specs=[
            pl.BlockSpec((NC, ROW_BLOCK, D), lambda i: (0, i, 0)),
            pl.BlockSpec((ROW_BLOCK, D), lambda i: (i, 0)),
            pl.BlockSpec((NC, ROW_BLOCK, 16), lambda i: (0, i, 0)),
            pl.BlockSpec((1, D), lambda i: (0, 0)),
            pl.BlockSpec((1, D), lambda i: (0, 0)),
            pl.BlockSpec((1, D), lambda i: (0, 0)),
        ],
        out_specs=pl.BlockSpec((ROW_BLOCK, D), lambda i: (i, 0)),
        out_shape=jax.ShapeDtypeStruct((N, D), jnp.float32),
    )(part, g, deg, b, gamma, beta)


def kernel(x, edge_index, W, b, gamma, beta):
    row32 = edge_index[0].astype(jnp.int32)
    col32 = edge_index[1].astype(jnp.int32)
    assert N_CHUNKS == NW * NCH and E_PAD == A_CHUNKS * ACH
    npad = E_PAD - E
    row_pad = jnp.concatenate(
        [row32, jnp.arange(npad, dtype=jnp.int32) % N])
    col_pad = jnp.concatenate(
        [col32, N + (jnp.arange(npad, dtype=jnp.int32) % ACH)])
    deg = _sc_degree(col32.reshape(N_CHUNKS, CHUNK))
    g = _tc_linear(deg, x, W)
    part = _sc_aggregate(g, row_pad.reshape(A_CHUNKS, ACH),
                         col_pad.reshape(A_CHUNKS, ACH))
    return _tc_final(part, g, deg, b.reshape(1, D), gamma.reshape(1, D),
                     beta.reshape(1, D))


# speedup vs baseline: 1.4276x; 1.0011x over previous
import functools

import jax
import jax.numpy as jnp
from jax import lax
from jax.experimental import pallas as pl
from jax.experimental.pallas import tpu as pltpu
from jax.experimental.pallas import tpu_sc as plsc

N = 10000
E = 320000
D = 128
BN_EPS = 1e-5

NC = 2
NS = 16
NW = NC * NS
CHUNK = 100
N_CHUNKS = E // CHUNK
NCH = N_CHUNKS // NW
ACH = 128
ANCH = 79
A_CHUNKS = NW * ANCH
E_PAD = A_CHUNKS * ACH
NBUF = 8
ROWS_PER_TILE = N // NS

_mesh = plsc.VectorSubcoreMesh(core_axis_name="c", subcore_axis_name="s")
_sc_params = pltpu.CompilerParams(use_tc_tiling_on_sc=False)


def _worker_id():
    cid = lax.axis_index("c")
    sid = lax.axis_index("s")
    return sid * NC + cid, cid, sid


@functools.partial(
    pl.kernel,
    out_type=jax.ShapeDtypeStruct((NC, N, 16), jnp.float32),
    mesh=_mesh,
    compiler_params=_sc_params,
    scratch_types=[
        pltpu.VMEM((NCH, CHUNK), jnp.int32),
        pltpu.VMEM((CHUNK, 16), jnp.float32),
        pltpu.VMEM((ROWS_PER_TILE, 16), jnp.float32),
        pltpu.VMEM_SHARED((N, 16), jnp.float32),
        [pltpu.SemaphoreType.DMA for _ in range(4)],
    ],
)
def _sc_degree(col2d_hbm, deg_hbm, cbuf, ones_v, zeros_v, acc_sh, ds_sems):
    wid, cid, sid = _worker_id()
    pltpu.sync_copy(col2d_hbm.at[pl.ds(wid * NCH, NCH)], cbuf)

    def fill(i, _):
        ones_v[i, :] = jnp.ones((16,), jnp.float32)
        return 0

    lax.fori_loop(0, CHUNK, fill, 0)

    def zfill(i, _):
        zeros_v[i, :] = jnp.zeros((16,), jnp.float32)
        return 0

    lax.fori_loop(0, ROWS_PER_TILE, zfill, 0)
    pltpu.sync_copy(zeros_v, acc_sh.at[pl.ds(sid * ROWS_PER_TILE, ROWS_PER_TILE)])
    plsc.subcore_barrier()

    def dscat(k, b):
        return pltpu.async_copy(ones_v, acc_sh.at[cbuf.at[k]], ds_sems[b],
                                add=True)

    def dscat_wait(k, b):
        pltpu.make_async_copy(ones_v, acc_sh.at[cbuf.at[k]], ds_sems[b]).wait()

    for j in range(4):
        dscat(j, j)

    def body(i, _):
        for off in range(4):
            j = 4 * i + off
            dscat_wait(j - 4, off)
            dscat(j, off)
        return 0

    lax.fori_loop(1, NCH // 4, body, 0)
    for off in range(4):
        dscat_wait(NCH - 4 + off, off)
    plsc.subcore_barrier()
    pltpu.sync_copy(
        acc_sh.at[pl.ds(sid * ROWS_PER_TILE, ROWS_PER_TILE)],
        deg_hbm.at[cid, pl.ds(sid * ROWS_PER_TILE, ROWS_PER_TILE)],
    )


@functools.partial(
    pl.kernel,
    out_type=jax.ShapeDtypeStruct((NC, N, D), jnp.bfloat16),
    mesh=_mesh,
    compiler_params=_sc_params,
    scratch_types=[
        pltpu.VMEM((ANCH, ACH), jnp.int32),
        pltpu.VMEM((ANCH, ACH), jnp.int32),
        [pltpu.VMEM((ACH, D), jnp.bfloat16) for _ in range(NBUF)],
        pltpu.VMEM_SHARED((N + ACH, D), jnp.bfloat16),
        [pltpu.SemaphoreType.DMA for _ in range(NBUF)],
        [pltpu.SemaphoreType.DMA for _ in range(NBUF)],
    ],
)
def _sc_aggregate(g_hbm, row2d_hbm, col2d_hbm, out_hbm,
                  rbuf, cbuf, rows, acc_sh, sg, ss):
    wid, cid, sid = _worker_id()
    pltpu.sync_copy(row2d_hbm.at[pl.ds(wid * ANCH, ANCH)], rbuf)
    pltpu.sync_copy(col2d_hbm.at[pl.ds(wid * ANCH, ANCH)], cbuf)

    def zfill(i, _):
        for j in range(D // 32):
            rows[0][i, pl.ds(j * 32, 32)] = jnp.zeros((32,), jnp.bfloat16)
        return 0

    lax.fori_loop(0, ACH, zfill, 0)

    def zcopy(k, _):
        pltpu.sync_copy(
            rows[0], acc_sh.at[pl.ds(sid * ROWS_PER_TILE + k * ACH, ACH)]
        )
        return 0

    lax.fori_loop(0, ROWS_PER_TILE // ACH, zcopy, 0)
    pltpu.sync_copy(
        rows[0].at[pl.ds(0, ROWS_PER_TILE % ACH)],
        acc_sh.at[pl.ds(sid * ROWS_PER_TILE + (ROWS_PER_TILE // ACH) * ACH,
                        ROWS_PER_TILE % ACH)],
    )
    plsc.subcore_barrier()

    def gather(k, b):
        pltpu.async_copy(g_hbm.at[rbuf.at[k]], rows[b], sg[b])

    def gather_wait(k, b):
        pltpu.make_async_copy(g_hbm.at[rbuf.at[k]], rows[b], sg[b]).wait()

    def scatter(k, b):
        pltpu.async_copy(rows[b], acc_sh.at[cbuf.at[k]], ss[b], add=True)

    def scatter_wait(k, b):
        pltpu.make_async_copy(rows[b], acc_sh.at[cbuf.at[k]], ss[b]).wait()

    LAG = NBUF - 2

    def slot(j, b):
        gather_wait(j, b)
        scatter(j, b)
        if isinstance(j, int):
            if j >= LAG:
                scatter_wait(j - LAG, (b - LAG) % NBUF)
            if j + 2 < ANCH:
                gather(j + 2, (b + 2) % NBUF)
        else:
            scatter_wait(j - LAG, (b - LAG) % NBUF)
            gather(j + 2, (b + 2) % NBUF)

    gather(0, 0)
    gather(1, 1)
    for j in range(LAG):
        slot(j, j)

    def octet(i, _):
        for off in range(NBUF):
            j = LAG + NBUF * i + off
            slot(j, (LAG + off) % NBUF)
        return 0

    _n_loop = (ANCH - 2 - LAG) // NBUF
    lax.fori_loop(0, _n_loop, octet, 0)
    for j in range(LAG + NBUF * _n_loop, ANCH):
        slot(j, j % NBUF)
    for k in range(LAG):
        scatter_wait(ANCH - LAG + k, (ANCH - LAG + k) % NBUF)
    plsc.subcore_barrier()
    pltpu.sync_copy(
        acc_sh.at[pl.ds(sid * ROWS_PER_TILE, ROWS_PER_TILE)],
        out_hbm.at[cid, pl.ds(sid * ROWS_PER_TILE, ROWS_PER_TILE)],
    )


ROW_BLOCK = 1000


def _tc_linear_body(deg_ref, x_ref, w_ref, g_ref):
    deg = deg_ref[0, :, 0] + deg_ref[1, :, 0] + 1.0
    dinv = lax.rsqrt(deg)
    h = jnp.dot(x_ref[...], w_ref[...], preferred_element_type=jnp.float32)
    g_ref[...] = (h * dinv[:, None]).astype(jnp.bfloat16)


def _tc_linear(deg, x, W):
    grid = N // ROW_BLOCK
    return pl.pallas_call(
        _tc_linear_body,
        grid=(grid,),
        in_specs=[
            pl.BlockSpec((NC, ROW_BLOCK, 16), lambda i: (0, i, 0)),
            pl.BlockSpec((ROW_BLOCK, D), lambda i: (i, 0)),
            pl.BlockSpec((D, D), lambda i: (0, 0)),
        ],
        out_specs=pl.BlockSpec((ROW_BLOCK, D), lambda i: (i, 0)),
        out_shape=jax.ShapeDtypeStruct((N, D), jnp.bfloat16),
    )(deg, x, W)


def _tc_final_body(part_ref, g_ref, deg_ref, b_ref, gam_ref, bet_ref, o_ref):
    deg = deg_ref[0, :, 0] + deg_ref[1, :, 0] + 1.0
    dinv = lax.rsqrt(deg)
    s = (part_ref[0].astype(jnp.float32) + part_ref[1].astype(jnp.float32)
         + g_ref[...].astype(jnp.float32))
    scale = (1.0 / jnp.sqrt(1.0 + BN_EPS))
    o = (s * dinv[:, None] + b_ref[0]) * (gam_ref[0] * scale) + bet_ref[0]
    o_ref[...] = jnp.maximum(o, 0.0)


def _tc_final(part, g, deg, b, gamma, beta):
    grid = N // ROW_BLOCK
    return pl.pallas_call(
        _tc_final_body,
        grid=(grid,),
        in_specs=[
            pl.BlockSpec((NC, ROW_BLOCK, D), lambda i: (0, i, 0)),
            pl.BlockSpec((ROW_BLOCK, D), lambda i: (i, 0)),
            pl.BlockSpec((NC, ROW_BLOCK, 16), lambda i: (0, i, 0)),
            pl.BlockSpec((1, D), lambda i: (0, 0)),
            pl.BlockSpec((1, D), lambda i: (0, 0)),
            pl.BlockSpec((1, D), lambda i: (0, 0)),
        ],
        out_specs=pl.BlockSpec((ROW_BLOCK, D), lambda i: (i, 0)),
        out_shape=jax.ShapeDtypeStruct((N, D), jnp.float32),
    )(part, g, deg, b, gamma, beta)


def kernel(x, edge_index, W, b, gamma, beta):
    row32 = edge_index[0].astype(jnp.int32)
    col32 = edge_index[1].astype(jnp.int32)
    assert N_CHUNKS == NW * NCH and E_PAD == A_CHUNKS * ACH
    npad = E_PAD - E
    row_pad = jnp.concatenate(
        [row32, jnp.arange(npad, dtype=jnp.int32) % N])
    col_pad = jnp.concatenate(
        [col32, N + (jnp.arange(npad, dtype=jnp.int32) % ACH)])
    deg = _sc_degree(col32.reshape(N_CHUNKS, CHUNK))
    g = _tc_linear(deg, x, W)
    part = _sc_aggregate(g, row_pad.reshape(A_CHUNKS, ACH),
                         col_pad.reshape(A_CHUNKS, ACH))
    return _tc_final(part, g, deg, b.reshape(1, D), gamma.reshape(1, D),
                     beta.reshape(1, D))


# speedup vs baseline: 1.4281x; 1.0004x over previous
import functools

import jax
import jax.numpy as jnp
from jax import lax
from jax.experimental import pallas as pl
from jax.experimental.pallas import tpu as pltpu
from jax.experimental.pallas import tpu_sc as plsc

N = 10000
E = 320000
D = 128
BN_EPS = 1e-5

NC = 2
NS = 16
NW = NC * NS
CHUNK = 100
N_CHUNKS = E // CHUNK
NCH = N_CHUNKS // NW
ACH = 128
ANCH = 79
A_CHUNKS = NW * ANCH
E_PAD = A_CHUNKS * ACH
NBUF = 4
ROWS_PER_TILE = N // NS

_mesh = plsc.VectorSubcoreMesh(core_axis_name="c", subcore_axis_name="s")
_sc_params = pltpu.CompilerParams(use_tc_tiling_on_sc=False)


def _worker_id():
    cid = lax.axis_index("c")
    sid = lax.axis_index("s")
    return sid * NC + cid, cid, sid


@functools.partial(
    pl.kernel,
    out_type=jax.ShapeDtypeStruct((NC, N, 16), jnp.float32),
    mesh=_mesh,
    compiler_params=_sc_params,
    scratch_types=[
        pltpu.VMEM((NCH, CHUNK), jnp.int32),
        pltpu.VMEM((CHUNK, 16), jnp.float32),
        pltpu.VMEM((ROWS_PER_TILE, 16), jnp.float32),
        pltpu.VMEM_SHARED((N, 16), jnp.float32),
        [pltpu.SemaphoreType.DMA for _ in range(4)],
    ],
)
def _sc_degree(col2d_hbm, deg_hbm, cbuf, ones_v, zeros_v, acc_sh, ds_sems):
    wid, cid, sid = _worker_id()
    pltpu.sync_copy(col2d_hbm.at[pl.ds(wid * NCH, NCH)], cbuf)

    def fill(i, _):
        ones_v[i, :] = jnp.ones((16,), jnp.float32)
        return 0

    lax.fori_loop(0, CHUNK, fill, 0)

    def zfill(i, _):
        zeros_v[i, :] = jnp.zeros((16,), jnp.float32)
        return 0

    lax.fori_loop(0, ROWS_PER_TILE, zfill, 0)
    pltpu.sync_copy(zeros_v, acc_sh.at[pl.ds(sid * ROWS_PER_TILE, ROWS_PER_TILE)])
    plsc.subcore_barrier()

    def dscat(k, b):
        return pltpu.async_copy(ones_v, acc_sh.at[cbuf.at[k]], ds_sems[b],
                                add=True)

    def dscat_wait(k, b):
        pltpu.make_async_copy(ones_v, acc_sh.at[cbuf.at[k]], ds_sems[b]).wait()

    for j in range(4):
        dscat(j, j)

    def body(i, _):
        for off in range(4):
            j = 4 * i + off
            dscat_wait(j - 4, off)
            dscat(j, off)
        return 0

    lax.fori_loop(1, NCH // 4, body, 0)
    for off in range(4):
        dscat_wait(NCH - 4 + off, off)
    plsc.subcore_barrier()
    pltpu.sync_copy(
        acc_sh.at[pl.ds(sid * ROWS_PER_TILE, ROWS_PER_TILE)],
        deg_hbm.at[cid, pl.ds(sid * ROWS_PER_TILE, ROWS_PER_TILE)],
    )


@functools.partial(
    pl.kernel,
    out_type=jax.ShapeDtypeStruct((NC, N, D), jnp.bfloat16),
    mesh=_mesh,
    compiler_params=_sc_params,
    scratch_types=[
        pltpu.VMEM((ANCH, ACH), jnp.int32),
        pltpu.VMEM((ANCH, ACH), jnp.int32),
        [pltpu.VMEM((ACH, D), jnp.bfloat16) for _ in range(NBUF)],
        pltpu.VMEM_SHARED((N + ACH, D), jnp.bfloat16),
        [pltpu.SemaphoreType.DMA for _ in range(NBUF)],
        [pltpu.SemaphoreType.DMA for _ in range(NBUF)],
    ],
)
def _sc_aggregate(g_hbm, row2d_hbm, col2d_hbm, out_hbm,
                  rbuf, cbuf, rows, acc_sh, sg, ss):
    wid, cid, sid = _worker_id()
    pltpu.sync_copy(row2d_hbm.at[pl.ds(wid * ANCH, ANCH)], rbuf)
    pltpu.sync_copy(col2d_hbm.at[pl.ds(wid * ANCH, ANCH)], cbuf)

    def zfill(i, _):
        for j in range(D // 32):
            rows[0][i, pl.ds(j * 32, 32)] = jnp.zeros((32,), jnp.bfloat16)
        return 0

    lax.fori_loop(0, ACH, zfill, 0)

    def zcopy(k, _):
        pltpu.sync_copy(
            rows[0], acc_sh.at[pl.ds(sid * ROWS_PER_TILE + k * ACH, ACH)]
        )
        return 0

    lax.fori_loop(0, ROWS_PER_TILE // ACH, zcopy, 0)
    pltpu.sync_copy(
        rows[0].at[pl.ds(0, ROWS_PER_TILE % ACH)],
        acc_sh.at[pl.ds(sid * ROWS_PER_TILE + (ROWS_PER_TILE // ACH) * ACH,
                        ROWS_PER_TILE % ACH)],
    )
    plsc.subcore_barrier()

    def gather(k, b):
        pltpu.async_copy(g_hbm.at[rbuf.at[k]], rows[b], sg[b])

    def gather_wait(k, b):
        pltpu.make_async_copy(g_hbm.at[rbuf.at[k]], rows[b], sg[b]).wait()

    def scatter(k, b):
        pltpu.async_copy(rows[b], acc_sh.at[cbuf.at[k]], ss[b], add=True)

    def scatter_wait(k, b):
        pltpu.make_async_copy(rows[b], acc_sh.at[cbuf.at[k]], ss[b]).wait()

    LAG = NBUF - 2

    def slot(j, b):
        gather_wait(j, b)
        scatter(j, b)
        if isinstance(j, int):
            if j >= LAG:
                scatter_wait(j - LAG, (b - LAG) % NBUF)
            if j + 2 < ANCH:
                gather(j + 2, (b + 2) % NBUF)
        else:
            scatter_wait(j - LAG, (b - LAG) % NBUF)
            gather(j + 2, (b + 2) % NBUF)

    gather(0, 0)
    gather(1, 1)
    for j in range(LAG):
        slot(j, j)

    def octet(i, _):
        for off in range(NBUF):
            j = LAG + NBUF * i + off
            slot(j, (LAG + off) % NBUF)
        return 0

    _n_loop = (ANCH - 2 - LAG) // NBUF
    lax.fori_loop(0, _n_loop, octet, 0)
    for j in range(LAG + NBUF * _n_loop, ANCH):
        slot(j, j % NBUF)
    for k in range(LAG):
        scatter_wait(ANCH - LAG + k, (ANCH - LAG + k) % NBUF)
    plsc.subcore_barrier()
    pltpu.sync_copy(
        acc_sh.at[pl.ds(sid * ROWS_PER_TILE, ROWS_PER_TILE)],
        out_hbm.at[cid, pl.ds(sid * ROWS_PER_TILE, ROWS_PER_TILE)],
    )


ROW_BLOCK = 1000


def _tc_linear_body(deg_ref, x_ref, w_ref, g_ref):
    deg = deg_ref[0, :, 0] + deg_ref[1, :, 0] + 1.0
    dinv = lax.rsqrt(deg)
    h = jnp.dot(x_ref[...], w_ref[...], preferred_element_type=jnp.float32)
    g_ref[...] = (h * dinv[:, None]).astype(jnp.bfloat16)


def _tc_linear(deg, x, W):
    grid = N // ROW_BLOCK
    return pl.pallas_call(
        _tc_linear_body,
        grid=(grid,),
        in_specs=[
            pl.BlockSpec((NC, ROW_BLOCK, 16), lambda i: (0, i, 0)),
            pl.BlockSpec((ROW_BLOCK, D), lambda i: (i, 0)),
            pl.BlockSpec((D, D), lambda i: (0, 0)),
        ],
        out_specs=pl.BlockSpec((ROW_BLOCK, D), lambda i: (i, 0)),
        out_shape=jax.ShapeDtypeStruct((N, D), jnp.bfloat16),
    )(deg, x, W)


def _tc_final_body(part_ref, g_ref, deg_ref, b_ref, gam_ref, bet_ref, o_ref):
    deg = deg_ref[0, :, 0] + deg_ref[1, :, 0] + 1.0
    dinv = lax.rsqrt(deg)
    s = (part_ref[0].astype(jnp.float32) + part_ref[1].astype(jnp.float32)
         + g_ref[...].astype(jnp.float32))
    scale = (1.0 / jnp.sqrt(1.0 + BN_EPS))
    o = (s * dinv[:, None] + b_ref[0]) * (gam_ref[0] * scale) + bet_ref[0]
    o_ref[...] = jnp.maximum(o, 0.0)


def _tc_final(part, g, deg, b, gamma, beta):
    grid = N // ROW_BLOCK
    return pl.pallas_call(
        _tc_final_body,
        grid=(grid,),
        in_specs=[
            pl.BlockSpec((NC, ROW_BLOCK, D), lambda i: (0, i, 0)),
            pl.BlockSpec((ROW_BLOCK, D), lambda i: (i, 0)),
            pl.BlockSpec((NC, ROW_BLOCK, 16), lambda i: (0, i, 0)),
            pl.BlockSpec((1, D), lambda i: (0, 0)),
            pl.BlockSpec((1, D), lambda i: (0, 0)),
            pl.BlockSpec((1, D), lambda i: (0, 0)),
        ],
        out_specs=pl.BlockSpec((ROW_BLOCK, D), lambda i: (i, 0)),
        out_shape=jax.ShapeDtypeStruct((N, D), jnp.float32),
    )(part, g, deg, b, gamma, beta)


def kernel(x, edge_index, W, b, gamma, beta):
    row32 = edge_index[0].astype(jnp.int32)
    col32 = edge_index[1].astype(jnp.int32)
    assert N_CHUNKS == NW * NCH and E_PAD == A_CHUNKS * ACH
    npad = E_PAD - E
    row_pad = jnp.concatenate(
        [row32, jnp.arange(npad, dtype=jnp.int32) % N])
    col_pad = jnp.concatenate(
        [col32, N + (jnp.arange(npad, dtype=jnp.int32) % ACH)])
    deg = _sc_degree(col32.reshape(N_CHUNKS, CHUNK))
    g = _tc_linear(deg, x, W)
    part = _sc_aggregate(g, row_pad.reshape(A_CHUNKS, ACH),
                         col_pad.reshape(A_CHUNKS, ACH))
    return _tc_final(part, g, deg, b.reshape(1, D), gamma.reshape(1, D),
                     beta.reshape(1, D))


# speedup vs baseline: 1.4306x; 1.0018x over previous
import functools

import jax
import jax.numpy as jnp
from jax import lax
from jax.experimental import pallas as pl
from jax.experimental.pallas import tpu as pltpu
from jax.experimental.pallas import tpu_sc as plsc

N = 10000
E = 320000
D = 128
BN_EPS = 1e-5

NC = 2
NS = 16
NW = NC * NS
CHUNK = 100
N_CHUNKS = E // CHUNK
NCH = N_CHUNKS // NW
ACH = 128
ANCH = 79
A_CHUNKS = NW * ANCH
E_PAD = A_CHUNKS * ACH
NBUF = 4
ROWS_PER_TILE = N // NS

_mesh = plsc.VectorSubcoreMesh(core_axis_name="c", subcore_axis_name="s",
                               num_cores=NC, num_subcores=NS)
_sc_params = pltpu.CompilerParams(use_tc_tiling_on_sc=False)


def _worker_id():
    cid = lax.axis_index("c")
    sid = lax.axis_index("s")
    return sid * NC + cid, cid, sid


@functools.partial(
    pl.kernel,
    out_type=jax.ShapeDtypeStruct((NC, N, 16), jnp.float32),
    mesh=_mesh,
    compiler_params=_sc_params,
    scratch_types=[
        pltpu.VMEM((NCH, CHUNK), jnp.int32),
        pltpu.VMEM((CHUNK, 16), jnp.float32),
        pltpu.VMEM((ROWS_PER_TILE, 16), jnp.float32),
        pltpu.VMEM_SHARED((N, 16), jnp.float32),
        [pltpu.SemaphoreType.DMA for _ in range(4)],
    ],
)
def _sc_degree(col2d_hbm, deg_hbm, cbuf, ones_v, zeros_v, acc_sh, ds_sems):
    wid, cid, sid = _worker_id()
    pltpu.sync_copy(col2d_hbm.at[pl.ds(wid * NCH, NCH)], cbuf)

    def fill(i, _):
        ones_v[i, :] = jnp.ones((16,), jnp.float32)
        return 0

    lax.fori_loop(0, CHUNK, fill, 0)

    def zfill(i, _):
        zeros_v[i, :] = jnp.zeros((16,), jnp.float32)
        return 0

    lax.fori_loop(0, ROWS_PER_TILE, zfill, 0)
    pltpu.sync_copy(zeros_v, acc_sh.at[pl.ds(sid * ROWS_PER_TILE, ROWS_PER_TILE)])
    plsc.subcore_barrier()

    def dscat(k, b):
        return pltpu.async_copy(ones_v, acc_sh.at[cbuf.at[k]], ds_sems[b],
                                add=True)

    def dscat_wait(k, b):
        pltpu.make_async_copy(ones_v, acc_sh.at[cbuf.at[k]], ds_sems[b]).wait()

    for j in range(4):
        dscat(j, j)

    def body(i, _):
        for off in range(4):
            j = 4 * i + off
            dscat_wait(j - 4, off)
            dscat(j, off)
        return 0

    lax.fori_loop(1, NCH // 4, body, 0)
    for off in range(4):
        dscat_wait(NCH - 4 + off, off)
    plsc.subcore_barrier()
    pltpu.sync_copy(
        acc_sh.at[pl.ds(sid * ROWS_PER_TILE, ROWS_PER_TILE)],
        deg_hbm.at[cid, pl.ds(sid * ROWS_PER_TILE, ROWS_PER_TILE)],
    )


@functools.partial(
    pl.kernel,
    out_type=jax.ShapeDtypeStruct((NC, N, D), jnp.bfloat16),
    mesh=_mesh,
    compiler_params=_sc_params,
    scratch_types=[
        pltpu.VMEM((ANCH, ACH), jnp.int32),
        pltpu.VMEM((ANCH, ACH), jnp.int32),
        [pltpu.VMEM((ACH, D), jnp.bfloat16) for _ in range(NBUF)],
        pltpu.VMEM_SHARED((N + ACH, D), jnp.bfloat16),
        [pltpu.SemaphoreType.DMA for _ in range(NBUF)],
        [pltpu.SemaphoreType.DMA for _ in range(NBUF)],
    ],
)
def _sc_aggregate(g_hbm, row2d_hbm, col2d_hbm, out_hbm,
                  rbuf, cbuf, rows, acc_sh, sg, ss):
    wid, cid, sid = _worker_id()
    pltpu.sync_copy(row2d_hbm.at[pl.ds(wid * ANCH, ANCH)], rbuf)
    pltpu.sync_copy(col2d_hbm.at[pl.ds(wid * ANCH, ANCH)], cbuf)

    def zfill(i, _):
        for j in range(D // 32):
            rows[0][i, pl.ds(j * 32, 32)] = jnp.zeros((32,), jnp.bfloat16)
        return 0

    lax.fori_loop(0, ACH, zfill, 0)

    def zcopy(k, _):
        pltpu.sync_copy(
            rows[0], acc_sh.at[pl.ds(sid * ROWS_PER_TILE + k * ACH, ACH)]
        )
        return 0

    lax.fori_loop(0, ROWS_PER_TILE // ACH, zcopy, 0)
    pltpu.sync_copy(
        rows[0].at[pl.ds(0, ROWS_PER_TILE % ACH)],
        acc_sh.at[pl.ds(sid * ROWS_PER_TILE + (ROWS_PER_TILE // ACH) * ACH,
                        ROWS_PER_TILE % ACH)],
    )
    plsc.subcore_barrier()

    def gather(k, b):
        pltpu.async_copy(g_hbm.at[rbuf.at[k]], rows[b], sg[b])

    def gather_wait(k, b):
        pltpu.make_async_copy(g_hbm.at[rbuf.at[k]], rows[b], sg[b]).wait()

    def scatter(k, b):
        pltpu.async_copy(rows[b], acc_sh.at[cbuf.at[k]], ss[b], add=True)

    def scatter_wait(k, b):
        pltpu.make_async_copy(rows[b], acc_sh.at[cbuf.at[k]], ss[b]).wait()

    LAG = NBUF - 2

    def slot(j, b):
        gather_wait(j, b)
        scatter(j, b)
        if isinstance(j, int):
            if j >= LAG:
                scatter_wait(j - LAG, (b - LAG) % NBUF)
            if j + 2 < ANCH:
                gather(j + 2, (b + 2) % NBUF)
        else:
            scatter_wait(j - LAG, (b - LAG) % NBUF)
            gather(j + 2, (b + 2) % NBUF)

    gather(0, 0)
    gather(1, 1)
    for j in range(LAG):
        slot(j, j)

    def octet(i, _):
        for off in range(NBUF):
            j = LAG + NBUF * i + off
            slot(j, (LAG + off) % NBUF)
        return 0

    _n_loop = (ANCH - 2 - LAG) // NBUF
    lax.fori_loop(0, _n_loop, octet, 0)
    for j in range(LAG + NBUF * _n_loop, ANCH):
        slot(j, j % NBUF)
    for k in range(LAG):
        scatter_wait(ANCH - LAG + k, (ANCH - LAG + k) % NBUF)
    plsc.subcore_barrier()
    pltpu.sync_copy(
        acc_sh.at[pl.ds(sid * ROWS_PER_TILE, ROWS_PER_TILE)],
        out_hbm.at[cid, pl.ds(sid * ROWS_PER_TILE, ROWS_PER_TILE)],
    )


ROW_BLOCK = 1000


def _tc_linear_body(deg_ref, x_ref, w_ref, g_ref):
    deg = deg_ref[0, :, 0] + deg_ref[1, :, 0] + 1.0
    dinv = lax.rsqrt(deg)
    h = jnp.dot(x_ref[...], w_ref[...], preferred_element_type=jnp.float32)
    g_ref[...] = (h * dinv[:, None]).astype(jnp.bfloat16)


def _tc_linear(deg, x, W):
    grid = N // ROW_BLOCK
    return pl.pallas_call(
        _tc_linear_body,
        grid=(grid,),
        in_specs=[
            pl.BlockSpec((NC, ROW_BLOCK, 16), lambda i: (0, i, 0)),
            pl.BlockSpec((ROW_BLOCK, D), lambda i: (i, 0)),
            pl.BlockSpec((D, D), lambda i: (0, 0)),
        ],
        out_specs=pl.BlockSpec((ROW_BLOCK, D), lambda i: (i, 0)),
        out_shape=jax.ShapeDtypeStruct((N, D), jnp.bfloat16),
    )(deg, x, W)


def _tc_final_body(part_ref, g_ref, deg_ref, b_ref, gam_ref, bet_ref, o_ref):
    deg = deg_ref[0, :, 0] + deg_ref[1, :, 0] + 1.0
    dinv = lax.rsqrt(deg)
    s = (part_ref[0].astype(jnp.float32) + part_ref[1].astype(jnp.float32)
         + g_ref[...].astype(jnp.float32))
    scale = (1.0 / jnp.sqrt(1.0 + BN_EPS))
    o = (s * dinv[:, None] + b_ref[0]) * (gam_ref[0] * scale) + bet_ref[0]
    o_ref[...] = jnp.maximum(o, 0.0)


def _tc_final(part, g, deg, b, gamma, beta):
    grid = N // ROW_BLOCK
    return pl.pallas_call(
        _tc_final_body,
        grid=(grid,),
        in_specs=[
            pl.BlockSpec((NC, ROW_BLOCK, D), lambda i: (0, i, 0)),
            pl.BlockSpec((ROW_BLOCK, D), lambda i: (i, 0)),
            pl.BlockSpec((NC, ROW_BLOCK, 16), lambda i: (0, i, 0)),
            pl.BlockSpec((1, D), lambda i: (0, 0)),
            pl.BlockSpec((1, D), lambda i: (0, 0)),
            pl.BlockSpec((1, D), lambda i: (0, 0)),
        ],
        out_specs=pl.BlockSpec((ROW_BLOCK, D), lambda i: (i, 0)),
        out_shape=jax.ShapeDtypeStruct((N, D), jnp.float32),
    )(part, g, deg, b, gamma, beta)


def kernel(x, edge_index, W, b, gamma, beta):
    row32 = edge_index[0].astype(jnp.int32)
    col32 = edge_index[1].astype(jnp.int32)
    assert N_CHUNKS == NW * NCH and E_PAD == A_CHUNKS * ACH
    npad = E_PAD - E
    row_pad = jnp.concatenate(
        [row32, jnp.arange(npad, dtype=jnp.int32) % N])
    col_pad = jnp.concatenate(
        [col32, N + (jnp.arange(npad, dtype=jnp.int32) % ACH)])
    deg = _sc_degree(col32.reshape(N_CHUNKS, CHUNK))
    g = _tc_linear(deg, x, W)
    part = _sc_aggregate(g, row_pad.reshape(A_CHUNKS, ACH),
                         col_pad.reshape(A_CHUNKS, ACH))
    return _tc_final(part, g, deg, b.reshape(1, D), gamma.reshape(1, D),
                     beta.reshape(1, D))
